# baseline (speedup 1.0000x reference)
import dataclasses
import functools

import numpy as np
import jax
from jax import lax
import jax.numpy as jnp
from jax.experimental import pallas as pl
from jax.experimental.pallas import tpu as pltpu
from jax.experimental.pallas import tpu_sc as plsc

_L = 7
_R = _L // 2
_H = 512
_W = 512
_B = 4
_TH = 128

_H_SC = 64
_H_TC = _H - _H_SC

_NC, _NS = 2, 16
_NW = _NC * _NS
_RPW = (_B * _H_SC) // _NW
_SPB = _H_SC // _RPW
_BUF_W = 128 + _W + 128
_NV = _BUF_W // 16
_BUF_H = _RPW + 16


def _disc_offsets():
    offs = []
    for ey in range(-_R, _R + 1):
        for ex in range(-_R, _R + 1):
            if 0 < ey * ey + ex * ex <= _R * _R:
                offs.append((ey, ex))
    return offs

_OFFSETS_NC = _disc_offsets()


def _rings():
    rings = {}
    for ey in range(-_R, _R + 1):
        for ex in range(-_R, _R + 1):
            d2 = ey * ey + ex * ex
            if d2 <= _R * _R:
                rings.setdefault(d2, []).append((ey, ex))
    out = []
    for d2, offs in sorted(rings.items()):
        groups = {}
        for (ey, ex) in offs:
            groups.setdefault(abs(ey), set()).add(ex)
        out.append((d2, sorted((ady, sorted(dxs)) for ady, dxs in groups.items())))
    return out

_RINGS = _rings()


def _tc_body(le_ref, dk_ref, mask_ref, x_ref, o_ref, pad_ref):
    le = le_ref[pl.program_id(0), 0]
    hin = min(_H_TC + _R, _H)
    rad = jnp.abs(x_ref[0, 3, 0:hin, :]) * le

    @pl.when(pl.program_id(0) == 0)
    def _zero_borders():
        pad_ref[:, 0:_R, :] = jnp.zeros((4, _R, _W + 2 * _R), jnp.float32)
        pad_ref[:, :, 0:_R] = jnp.zeros((4, _R + hin, _R), jnp.float32)
        pad_ref[:, :, _R + _W:] = jnp.zeros((4, _R + hin, _R), jnp.float32)

    pad_ref[0, _R:_R + hin, _R:_R + _W] = x_ref[0, 0, 0:hin, :]
    pad_ref[1, _R:_R + hin, _R:_R + _W] = x_ref[0, 1, 0:hin, :]
    pad_ref[2, _R:_R + hin, _R:_R + _W] = x_ref[0, 2, 0:hin, :]
    pad_ref[3, _R:_R + hin, _R:_R + _W] = rad

    for y0 in range(0, _H_TC, _TH):
        th = min(_TH, _H_TC - y0)
        prad = pad_ref[3, y0:y0 + th + 2 * _R, :]
        prgb = [pad_ref[c, y0:y0 + th + 2 * _R, :] for c in range(3)]
        acc = [None, None, None, None]
        for d2, groups in _RINGS:
            rey, rex = next((ey, ex) for ey in range(-_R, _R + 1)
                            for ex in range(-_R, _R + 1)
                            if ey * ey + ex * ex == d2)
            d = dk_ref[_R - rey, _R - rex]
            m = mask_ref[_R - rey, _R - rex]
            w_pad = jnp.clip(prad - (d - 1.0), 0.0, 1.0) * m
            p = [w_pad * prgb[c] for c in range(3)]
            p.append(w_pad)
            for ady, dxs in groups:
                for ci in range(4):
                    xs = None
                    for dx in dxs:
                        t = p[ci][:, _R + dx:_R + dx + _W]
                        xs = t if xs is None else xs + t
                    for ey in ({0} if ady == 0 else {-ady, ady}):
                        t = xs[_R + ey:_R + ey + th, :]
                        acc[ci] = t if acc[ci] is None else acc[ci] + t

        inv = 1.0 / (acc[3] + 1e-8)
        o_ref[0, 0, y0:y0 + th, :] = acc[0] * inv
        o_ref[0, 1, y0:y0 + th, :] = acc[1] * inv
        o_ref[0, 2, y0:y0 + th, :] = acc[2] * inv


def _tc_render(x, lens_effects, diskernel, lens_mask):
    hin = min(_H_TC + _R, _H)
    return pl.pallas_call(
        _tc_body,
        grid=(_B,),
        in_specs=[
            pl.BlockSpec((_B, 1), lambda i: (0, 0), memory_space=pltpu.SMEM),
            pl.BlockSpec((_L, _L), lambda i: (0, 0), memory_space=pltpu.SMEM),
            pl.BlockSpec((_L, _L), lambda i: (0, 0), memory_space=pltpu.SMEM),
            pl.BlockSpec((1, 4, _H, _W), lambda i: (i, 0, 0, 0)),
        ],
        out_specs=pl.BlockSpec((1, 3, _H_TC, _W), lambda i: (i, 0, 0, 0)),
        out_shape=jax.ShapeDtypeStruct((_B, 3, _H_TC, _W), x.dtype),
        scratch_shapes=[pltpu.VMEM((4, hin + _R, _W + 2 * _R), jnp.float32)],
    )(lens_effects, diskernel, lens_mask, x)


_RING_D2S = [d2 for d2, _g in _RINGS if d2 != 0]


def _sc_render(x, lens_effects, diskernel):
    le_b = jnp.broadcast_to(lens_effects, (_B, 16))
    drep = []
    for d2 in _RING_D2S:
        rey, rex = next((ey, ex) for ey in range(-_R, _R + 1)
                        for ex in range(-_R, _R + 1)
                        if ey * ey + ex * ex == d2)
        drep.append(diskernel[_R - rey, _R - rex] - 1.0)
    dm1_b = jnp.broadcast_to(jnp.stack(drep)[:, None], (len(_RING_D2S), 16))

    mesh = plsc.VectorSubcoreMesh(core_axis_name="c", subcore_axis_name="s")
    cp = pltpu.CompilerParams()
    if "needs_layout_passes" in pltpu.CompilerParams.__dataclass_fields__:
        cp = dataclasses.replace(cp, needs_layout_passes=False)
    if "use_tc_tiling_on_sc" in pltpu.CompilerParams.__dataclass_fields__:
        cp = dataclasses.replace(cp, use_tc_tiling_on_sc=False)

    @functools.partial(
        pl.kernel,
        mesh=mesh,
        compiler_params=cp,
        out_type=jax.ShapeDtypeStruct((_B, 3, _H_SC, _W), jnp.float32),
        scratch_types=[
            pltpu.VMEM((4, _BUF_H, _BUF_W), jnp.float32),
            pltpu.VMEM((3, _RPW, _W), jnp.float32),
            pltpu.VMEM((_B, 16), jnp.float32),
            pltpu.VMEM((len(_RING_D2S), 16), jnp.float32),
            pltpu.SemaphoreType.DMA,
        ],
    )
    def k(x_hbm, le_hbm, dm1_hbm, out_hbm, buf, obuf, le_v, dm1_v, sem):
        wid = lax.axis_index("s") * _NC + lax.axis_index("c")
        b = wid // _SPB
        s = wid % _SPB
        r0 = _H_TC + s * _RPW
        lo = r0 - 8

        pltpu.sync_copy(le_hbm, le_v)
        pltpu.sync_copy(dm1_hbm, dm1_v)
        le = le_v[b, :]

        is_bot = s == (_SPB - 1)

        @pl.when(jnp.logical_not(is_bot))
        def _load_full():
            pltpu.async_copy(
                x_hbm.at[b, :, pl.ds(lo, _BUF_H), :],
                buf.at[:, :, 128:128 + _W], sem).wait()

        @pl.when(is_bot)
        def _load_clipped():
            nrow = _BUF_H - 8
            pltpu.async_copy(
                x_hbm.at[b, :, pl.ds(lo, nrow), :],
                buf.at[:, 0:nrow, 128:128 + _W], sem).wait()
            @pl.loop(0, 4)
            def _(ch):
                @pl.loop(_BUF_H - 8, _BUF_H)
                def _(r):
                    @pl.loop(0, _NV)
                    def _(v):
                        buf[ch, r, pl.ds(v * 16, 16)] = jnp.zeros((16,), jnp.float32)

        @pl.loop(0, 4)
        def _(ch):
            @pl.loop(0, _BUF_H)
            def _(r):
                buf[ch, r, pl.ds(112, 16)] = jnp.zeros((16,), jnp.float32)
                buf[ch, r, pl.ds(128 + _W, 16)] = jnp.zeros((16,), jnp.float32)

        @pl.loop(0, _BUF_H)
        def _(r):
            @pl.loop(7, 41)
            def _(v):
                sl = pl.ds(v * 16, 16)
                buf[3, r, sl] = jnp.abs(buf[3, r, sl]) * le

        dm1 = {d2: dm1_v[i, :] for i, d2 in enumerate(_RING_D2S)}

        iota16 = jnp.arange(16, dtype=jnp.int32)
        chv = [jnp.full((16,), c, jnp.int32) for c in range(4)]

        @pl.loop(0, _RPW)
        def _(y):
            rowvs = {ey: jnp.full((16,), y + 8 + ey, jnp.int32)
                     for ey in range(-_R, _R + 1)}

            @plsc.parallel_loop(0, _W // 16, unroll=2)
            def _(xv):
                c0 = xv * 16
                accr = buf[0, y + 8, pl.ds(c0 + 128, 16)]
                accg = buf[1, y + 8, pl.ds(c0 + 128, 16)]
                accb = buf[2, y + 8, pl.ds(c0 + 128, 16)]
                accw = jnp.full((16,), 1.0, jnp.float32)
                base = iota16 + (c0 + 128)
                colvs = {ex: base + ex for ex in range(-_R, _R + 1) if ex != 0}
                for (ey, ex) in _OFFSETS_NC:
                    row = y + 8 + ey
                    if ex == 0:
                        sl = pl.ds(c0 + 128, 16)
                        srad = buf[3, row, sl]
                        srgb = [buf[c, row, sl] for c in range(3)]
                    else:
                        rowv = rowvs[ey]
                        colv = colvs[ex]
                        srad = plsc.load_gather(buf, [chv[3], rowv, colv])
                        srgb = [plsc.load_gather(buf, [chv[c], rowv, colv])
                                for c in range(3)]
                    w = jnp.minimum(jnp.maximum(srad - dm1[ey * ey + ex * ex],
                                                0.0), 1.0)
                    accw = accw + w
                    accr = accr + w * srgb[0]
                    accg = accg + w * srgb[1]
                    accb = accb + w * srgb[2]
                inv = 1.0 / (accw + 1e-8)
                osl = pl.ds(c0, 16)
                obuf[0, y, osl] = accr * inv
                obuf[1, y, osl] = accg * inv
                obuf[2, y, osl] = accb * inv

        pltpu.sync_copy(obuf, out_hbm.at[b, :, pl.ds(r0 - _H_TC, _RPW), :])

    return k(x, le_b, dm1_b)


@jax.jit
def kernel(x, lens_effects, diskernel, lens_mask):
    out_tc = _tc_render(x, lens_effects, diskernel, lens_mask)
    out_sc = _sc_render(x, lens_effects, diskernel)
    return jnp.concatenate([out_tc, out_sc], axis=2)


# speedup vs baseline: 1.0522x; 1.0522x over previous
import dataclasses
import functools

import numpy as np
import jax
from jax import lax
import jax.numpy as jnp
from jax.experimental import pallas as pl
from jax.experimental.pallas import tpu as pltpu
from jax.experimental.pallas import tpu_sc as plsc

_L = 7
_R = _L // 2
_H = 512
_W = 512
_B = 4
_TH = 128

_H_SC = 64
_H_TC = _H - _H_SC

_NC, _NS = 2, 16
_NW = _NC * _NS
_RPW = (_B * _H_SC) // _NW
_SPB = _H_SC // _RPW
_BUF_W = 128 + _W + 128
_NV = _BUF_W // 16
_BUF_H = _RPW + 16


def _disc_offsets():
    offs = []
    for ey in range(-_R, _R + 1):
        for ex in range(-_R, _R + 1):
            if 0 < ey * ey + ex * ex <= _R * _R:
                offs.append((ey, ex))
    return offs

_OFFSETS_NC = _disc_offsets()


def _rings():
    rings = {}
    for ey in range(-_R, _R + 1):
        for ex in range(-_R, _R + 1):
            d2 = ey * ey + ex * ex
            if d2 <= _R * _R:
                rings.setdefault(d2, []).append((ey, ex))
    out = []
    for d2, offs in sorted(rings.items()):
        groups = {}
        for (ey, ex) in offs:
            groups.setdefault(abs(ey), set()).add(ex)
        out.append((d2, sorted((ady, sorted(dxs)) for ady, dxs in groups.items())))
    return out

_RINGS = _rings()


def _tc_body(le_ref, dk_ref, mask_ref, x_ref, o_ref, pad_ref):
    le = le_ref[pl.program_id(0), 0]
    hin = min(_H_TC + _R, _H)
    rad = jnp.abs(x_ref[0, 3, 0:hin, :]) * le

    @pl.when(pl.program_id(0) == 0)
    def _zero_borders():
        pad_ref[:, 0:_R, :] = jnp.zeros((4, _R, _W + 2 * _R), jnp.float32)
        pad_ref[:, :, 0:_R] = jnp.zeros((4, _R + hin, _R), jnp.float32)
        pad_ref[:, :, _R + _W:] = jnp.zeros((4, _R + hin, _R), jnp.float32)

    pad_ref[0, _R:_R + hin, _R:_R + _W] = x_ref[0, 0, 0:hin, :]
    pad_ref[1, _R:_R + hin, _R:_R + _W] = x_ref[0, 1, 0:hin, :]
    pad_ref[2, _R:_R + hin, _R:_R + _W] = x_ref[0, 2, 0:hin, :]
    pad_ref[3, _R:_R + hin, _R:_R + _W] = rad

    for y0 in range(0, _H_TC, _TH):
        th = min(_TH, _H_TC - y0)
        prad = pad_ref[3, y0:y0 + th + 2 * _R, :]
        prgb = [pad_ref[c, y0:y0 + th + 2 * _R, :] for c in range(3)]
        acc = [None, None, None, None]
        for d2, groups in _RINGS:
            rey, rex = next((ey, ex) for ey in range(-_R, _R + 1)
                            for ex in range(-_R, _R + 1)
                            if ey * ey + ex * ex == d2)
            d = dk_ref[_R - rey, _R - rex]
            m = mask_ref[_R - rey, _R - rex]
            w_pad = jnp.clip(prad - (d - 1.0), 0.0, 1.0) * m
            p = [w_pad * prgb[c] for c in range(3)]
            p.append(w_pad)
            for ady, dxs in groups:
                for ci in range(4):
                    xs = None
                    for dx in dxs:
                        t = p[ci][:, _R + dx:_R + dx + _W]
                        xs = t if xs is None else xs + t
                    for ey in ({0} if ady == 0 else {-ady, ady}):
                        t = xs[_R + ey:_R + ey + th, :]
                        acc[ci] = t if acc[ci] is None else acc[ci] + t

        inv = 1.0 / (acc[3] + 1e-8)
        o_ref[0, 0, y0:y0 + th, :] = acc[0] * inv
        o_ref[0, 1, y0:y0 + th, :] = acc[1] * inv
        o_ref[0, 2, y0:y0 + th, :] = acc[2] * inv


def _tc_render(x, lens_effects, diskernel, lens_mask):
    hin = min(_H_TC + _R, _H)
    return pl.pallas_call(
        _tc_body,
        grid=(_B,),
        in_specs=[
            pl.BlockSpec((_B, 1), lambda i: (0, 0), memory_space=pltpu.SMEM),
            pl.BlockSpec((_L, _L), lambda i: (0, 0), memory_space=pltpu.SMEM),
            pl.BlockSpec((_L, _L), lambda i: (0, 0), memory_space=pltpu.SMEM),
            pl.BlockSpec((1, 4, _H, _W), lambda i: (i, 0, 0, 0)),
        ],
        out_specs=pl.BlockSpec((1, 3, _H, _W), lambda i: (i, 0, 0, 0)),
        out_shape=jax.ShapeDtypeStruct((_B, 3, _H, _W), x.dtype),
        scratch_shapes=[pltpu.VMEM((4, hin + _R, _W + 2 * _R), jnp.float32)],
    )(lens_effects, diskernel, lens_mask, x)


_RING_D2S = [d2 for d2, _g in _RINGS if d2 != 0]


def _sc_render(x, lens_effects, diskernel):
    le_b = jnp.broadcast_to(lens_effects, (_B, 16))
    drep = []
    for d2 in _RING_D2S:
        rey, rex = next((ey, ex) for ey in range(-_R, _R + 1)
                        for ex in range(-_R, _R + 1)
                        if ey * ey + ex * ex == d2)
        drep.append(diskernel[_R - rey, _R - rex] - 1.0)
    dm1_b = jnp.broadcast_to(jnp.stack(drep)[:, None], (len(_RING_D2S), 16))

    mesh = plsc.VectorSubcoreMesh(core_axis_name="c", subcore_axis_name="s")
    cp = pltpu.CompilerParams()
    if "needs_layout_passes" in pltpu.CompilerParams.__dataclass_fields__:
        cp = dataclasses.replace(cp, needs_layout_passes=False)
    if "use_tc_tiling_on_sc" in pltpu.CompilerParams.__dataclass_fields__:
        cp = dataclasses.replace(cp, use_tc_tiling_on_sc=False)

    @functools.partial(
        pl.kernel,
        mesh=mesh,
        compiler_params=cp,
        out_type=jax.ShapeDtypeStruct((_B, 3, _H_SC, _W), jnp.float32),
        scratch_types=[
            pltpu.VMEM((4, _BUF_H, _BUF_W), jnp.float32),
            pltpu.VMEM((3, _RPW, _W), jnp.float32),
            pltpu.VMEM((_B, 16), jnp.float32),
            pltpu.VMEM((len(_RING_D2S), 16), jnp.float32),
            pltpu.SemaphoreType.DMA,
        ],
    )
    def k(x_hbm, le_hbm, dm1_hbm, out_hbm, buf, obuf, le_v, dm1_v, sem):
        wid = lax.axis_index("s") * _NC + lax.axis_index("c")
        b = wid // _SPB
        s = wid % _SPB
        r0 = _H_TC + s * _RPW
        lo = r0 - 8

        pltpu.sync_copy(le_hbm, le_v)
        pltpu.sync_copy(dm1_hbm, dm1_v)
        le = le_v[b, :]

        is_bot = s == (_SPB - 1)

        @pl.when(jnp.logical_not(is_bot))
        def _load_full():
            pltpu.async_copy(
                x_hbm.at[b, :, pl.ds(lo, _BUF_H), :],
                buf.at[:, :, 128:128 + _W], sem).wait()

        @pl.when(is_bot)
        def _load_clipped():
            nrow = _BUF_H - 8
            pltpu.async_copy(
                x_hbm.at[b, :, pl.ds(lo, nrow), :],
                buf.at[:, 0:nrow, 128:128 + _W], sem).wait()
            @pl.loop(0, 4)
            def _(ch):
                @pl.loop(_BUF_H - 8, _BUF_H)
                def _(r):
                    @pl.loop(0, _NV)
                    def _(v):
                        buf[ch, r, pl.ds(v * 16, 16)] = jnp.zeros((16,), jnp.float32)

        @pl.loop(0, 4)
        def _(ch):
            @pl.loop(0, _BUF_H)
            def _(r):
                buf[ch, r, pl.ds(112, 16)] = jnp.zeros((16,), jnp.float32)
                buf[ch, r, pl.ds(128 + _W, 16)] = jnp.zeros((16,), jnp.float32)

        @pl.loop(0, _BUF_H)
        def _(r):
            @pl.loop(7, 41)
            def _(v):
                sl = pl.ds(v * 16, 16)
                buf[3, r, sl] = jnp.abs(buf[3, r, sl]) * le

        dm1 = {d2: dm1_v[i, :] for i, d2 in enumerate(_RING_D2S)}

        iota16 = jnp.arange(16, dtype=jnp.int32)
        chv = [jnp.full((16,), c, jnp.int32) for c in range(4)]

        @pl.loop(0, _RPW)
        def _(y):
            rowvs = {ey: jnp.full((16,), y + 8 + ey, jnp.int32)
                     for ey in range(-_R, _R + 1)}

            @plsc.parallel_loop(0, _W // 16, unroll=2)
            def _(xv):
                c0 = xv * 16
                accr = buf[0, y + 8, pl.ds(c0 + 128, 16)]
                accg = buf[1, y + 8, pl.ds(c0 + 128, 16)]
                accb = buf[2, y + 8, pl.ds(c0 + 128, 16)]
                accw = jnp.full((16,), 1.0, jnp.float32)
                base = iota16 + (c0 + 128)
                colvs = {ex: base + ex for ex in range(-_R, _R + 1) if ex != 0}
                for (ey, ex) in _OFFSETS_NC:
                    row = y + 8 + ey
                    if ex == 0:
                        sl = pl.ds(c0 + 128, 16)
                        srad = buf[3, row, sl]
                        srgb = [buf[c, row, sl] for c in range(3)]
                    else:
                        rowv = rowvs[ey]
                        colv = colvs[ex]
                        srad = plsc.load_gather(buf, [chv[3], rowv, colv])
                        srgb = [plsc.load_gather(buf, [chv[c], rowv, colv])
                                for c in range(3)]
                    w = jnp.minimum(jnp.maximum(srad - dm1[ey * ey + ex * ex],
                                                0.0), 1.0)
                    accw = accw + w
                    accr = accr + w * srgb[0]
                    accg = accg + w * srgb[1]
                    accb = accb + w * srgb[2]
                inv = 1.0 / (accw + 1e-8)
                osl = pl.ds(c0, 16)
                obuf[0, y, osl] = accr * inv
                obuf[1, y, osl] = accg * inv
                obuf[2, y, osl] = accb * inv

        pltpu.sync_copy(obuf, out_hbm.at[b, :, pl.ds(r0 - _H_TC, _RPW), :])

    return k(x, le_b, dm1_b)


@jax.jit
def kernel(x, lens_effects, diskernel, lens_mask):
    out_tc = _tc_render(x, lens_effects, diskernel, lens_mask)
    out_sc = _sc_render(x, lens_effects, diskernel)
    return jax.lax.dynamic_update_slice(out_tc, out_sc, (0, 0, _H_TC, 0))
